# baseline (speedup 1.0000x reference)
import functools

import jax
import jax.numpy as jnp
from jax.experimental import pallas as pl

GRID = 10
N = GRID * GRID
H = 256
L = 6


def _gnn_body(xt_ref, wenc_ref, benc_ref, wf_ref, bdir_ref, w2u_ref,
              cu_ref, u2_ref, cc_ref, gam_ref,
              d1w_ref, d1b_ref, d2w_ref, d2b_ref, out_ref):
    bc = xt_ref.shape[1]
    rows = N * bc
    f32 = jnp.float32

    def mm(a, b):
        return jax.lax.dot_general(a, b, (((1,), (0,)), ((), ())),
                                   preferred_element_type=f32)

    node = jax.lax.broadcasted_iota(jnp.int32, (rows, 1), 0) // bc
    colv = node % GRID
    rowv = node // GRID
    deg = ((rowv >= 1).astype(f32) + (rowv <= GRID - 2).astype(f32)
           + (colv >= 1).astype(f32) + (colv <= GRID - 2).astype(f32))

    x = xt_ref[...].reshape(rows, 5)
    h = jnp.maximum(mm(x, wenc_ref[...]) + benc_ref[...], 0.0)

    zrow = jnp.zeros((GRID * bc, H), f32)
    zcol = jnp.zeros((GRID, 1, bc, H), f32)

    hn = h
    for l in range(L):
        aw = mm(hn, wf_ref[l])
        a = aw[:, :H]
        p = aw[:, H:]
        bd = bdir_ref[l]
        t0 = jnp.maximum(a + bd[0:1], 0.0)
        t1 = jnp.maximum(a + bd[1:2], 0.0)
        t2 = jnp.maximum(a + bd[2:3], 0.0).reshape(GRID, GRID, bc, H)
        t3 = jnp.maximum(a + bd[3:4], 0.0).reshape(GRID, GRID, bc, H)
        s = jnp.concatenate([zrow, t0[:rows - GRID * bc]], axis=0)
        s = s + jnp.concatenate([t1[GRID * bc:], zrow], axis=0)
        sc = jnp.concatenate([zcol, t2[:, :GRID - 1]], axis=1)
        sc = sc + jnp.concatenate([t3[:, 1:], zcol], axis=1)
        s = s + sc.reshape(rows, H)
        u = jnp.maximum(p + mm(s, w2u_ref[l]) + deg * cu_ref[l][0:1]
                        + cu_ref[l][1:2], 0.0)
        if l == 0:
            y = h + mm(u, u2_ref[l]) + cc_ref[l]
        else:
            y = hn * gam_ref[l - 1] + mm(u, u2_ref[l]) + cc_ref[l]
        mu = jnp.mean(y, axis=-1, keepdims=True)
        yc = y - mu
        var = jnp.mean(yc * yc, axis=-1, keepdims=True)
        hn = yc * jax.lax.rsqrt(var + 1e-5)

    g = jnp.maximum(mm(hn, d1w_ref[...]) + d1b_ref[...], 0.0)
    o = jnp.sum(g.reshape(N, bc, 32) * d2w_ref[...].reshape(1, 1, 32), axis=-1)
    o = o + d2b_ref[0, 0]
    out_ref[...] = jax.nn.sigmoid(o).reshape(1, N, bc)


@jax.jit
def kernel(x, edge_dir, W_enc, b_enc, W1, b1, W2, b2, U1, c1, U2, c2,
           gamma, beta, D1, d1, D2, d2, src_idx, dst_idx):
    B = x.shape[0]
    bc = 64
    while B % bc:
        bc //= 2

    xt = jnp.transpose(x, (1, 0, 2))
    w1h = W1[:, :H, :]
    wrow = W1[:, H, :]
    ks = jnp.arange(4, dtype=jnp.float32).reshape(1, 4, 1) / 3.0
    u1a = U1[:, :H, :]
    u1b = U1[:, H:, :]
    wfused = jnp.concatenate([w1h, u1a], axis=2)
    w2u = jnp.einsum('lij,ljk->lik', W2, u1b)
    b2u = jnp.einsum('lj,ljk->lk', b2, u1b)

    gscale = jnp.concatenate([jnp.ones((1, H), jnp.float32), gamma[:-1]], 0)
    bshift = jnp.concatenate([jnp.zeros((1, H), jnp.float32), beta[:-1]], 0)
    bfold = jnp.einsum('lj,ljk->lk', bshift, wfused)
    wfused = gscale[:, :, None] * wfused
    bdir = (b1[:, None, :] + ks * wrow[:, None, :]
            + bfold[:, None, :H])
    cu = jnp.stack([b2u, c1 + bfold[:, H:]], axis=1)
    ccomb = c2 + bshift
    d1g = gamma[-1][:, None] * D1
    d1f = (d1 + beta[-1] @ D1).reshape(1, 32)

    full = lambda *shape: pl.BlockSpec(shape, lambda i: (0,) * len(shape))
    out = pl.pallas_call(
        _gnn_body,
        grid=(B // bc,),
        in_specs=[
            pl.BlockSpec((N, bc, 5), lambda i: (0, i, 0)),
            full(5, H), full(1, H),
            full(L, H, 2 * H), full(L, 4, H), full(L, H, H),
            full(L, 2, H),
            full(L, H, H), full(L, 1, H), full(L - 1, 1, H),
            full(H, 32), full(1, 32), full(32, 1), full(1, 1),
        ],
        out_specs=pl.BlockSpec((1, N, bc), lambda i: (i, 0, 0)),
        out_shape=jax.ShapeDtypeStruct((B // bc, N, bc), jnp.float32),
    )(xt, W_enc, b_enc.reshape(1, H),
      wfused, bdir, w2u, cu,
      U2, ccomb[:, None, :], gamma[:-1][:, None, :],
      d1g, d1f, D2, d2.reshape(1, 1))
    return out.transpose(0, 2, 1).reshape(B, N)


# speedup vs baseline: 1.3734x; 1.3734x over previous
import functools

import jax
import jax.numpy as jnp
from jax.experimental import pallas as pl

GRID = 10
N = GRID * GRID
H = 256
L = 6


def _gnn_body(xt_ref, wenc_ref, benc_ref, wf_ref, bdir_ref, w2u_ref,
              cu_ref, u2_ref, cc_ref, gam_ref,
              d1w_ref, d1b_ref, d2w_ref, d2b_ref, out_ref):
    bc = xt_ref.shape[1]
    rows = N * bc
    f32 = jnp.float32

    def mm(a, b):
        return jax.lax.dot_general(a, b, (((1,), (0,)), ((), ())),
                                   preferred_element_type=f32)

    node = jax.lax.broadcasted_iota(jnp.int32, (rows, 1), 0) // bc
    colv = node % GRID
    rowv = node // GRID
    deg = ((rowv >= 1).astype(f32) + (rowv <= GRID - 2).astype(f32)
           + (colv >= 1).astype(f32) + (colv <= GRID - 2).astype(f32))

    x = xt_ref[...].reshape(rows, 5)
    h = jnp.maximum(mm(x, wenc_ref[...]) + benc_ref[...], 0.0)

    zrow = jnp.zeros((GRID * bc, H), f32)
    zcol = jnp.zeros((GRID, 1, bc, H), f32)

    hn = h
    for l in range(L):
        aw = mm(hn, wf_ref[l])
        a = aw[:, :H]
        p = aw[:, H:]
        bd = bdir_ref[l]
        t0 = jnp.maximum(a + bd[0:1], 0.0)
        t1 = jnp.maximum(a + bd[1:2], 0.0)
        t2 = jnp.maximum(a + bd[2:3], 0.0).reshape(GRID, GRID, bc, H)
        t3 = jnp.maximum(a + bd[3:4], 0.0).reshape(GRID, GRID, bc, H)
        s = jnp.concatenate([zrow, t0[:rows - GRID * bc]], axis=0)
        s = s + jnp.concatenate([t1[GRID * bc:], zrow], axis=0)
        sc = jnp.concatenate([zcol, t2[:, :GRID - 1]], axis=1)
        sc = sc + jnp.concatenate([t3[:, 1:], zcol], axis=1)
        s = s + sc.reshape(rows, H)
        u = jnp.maximum(p + mm(s, w2u_ref[l]) + deg * cu_ref[l][0:1]
                        + cu_ref[l][1:2], 0.0)
        if l == 0:
            y = h + mm(u, u2_ref[l]) + cc_ref[l]
        else:
            y = hn * gam_ref[l - 1] + mm(u, u2_ref[l]) + cc_ref[l]
        mu = jnp.mean(y, axis=-1, keepdims=True)
        yc = y - mu
        var = jnp.mean(yc * yc, axis=-1, keepdims=True)
        hn = yc * jax.lax.rsqrt(var + 1e-5)

    g = jnp.maximum(mm(hn, d1w_ref[...]) + d1b_ref[...], 0.0)
    o = jnp.sum(g.reshape(N, bc, 32) * d2w_ref[...].reshape(1, 1, 32), axis=-1)
    o = o + d2b_ref[0, 0]
    out_ref[...] = jax.nn.sigmoid(o).reshape(1, N, bc)


@jax.jit
def kernel(x, edge_dir, W_enc, b_enc, W1, b1, W2, b2, U1, c1, U2, c2,
           gamma, beta, D1, d1, D2, d2, src_idx, dst_idx):
    B = x.shape[0]
    bc = 16
    while B % bc:
        bc //= 2

    xt = jnp.transpose(x, (1, 0, 2))
    w1h = W1[:, :H, :]
    wrow = W1[:, H, :]
    ks = jnp.arange(4, dtype=jnp.float32).reshape(1, 4, 1) / 3.0
    u1a = U1[:, :H, :]
    u1b = U1[:, H:, :]
    wfused = jnp.concatenate([w1h, u1a], axis=2)
    w2u = jnp.einsum('lij,ljk->lik', W2, u1b)
    b2u = jnp.einsum('lj,ljk->lk', b2, u1b)

    gscale = jnp.concatenate([jnp.ones((1, H), jnp.float32), gamma[:-1]], 0)
    bshift = jnp.concatenate([jnp.zeros((1, H), jnp.float32), beta[:-1]], 0)
    bfold = jnp.einsum('lj,ljk->lk', bshift, wfused)
    wfused = gscale[:, :, None] * wfused
    bdir = (b1[:, None, :] + ks * wrow[:, None, :]
            + bfold[:, None, :H])
    cu = jnp.stack([b2u, c1 + bfold[:, H:]], axis=1)
    ccomb = c2 + bshift
    d1g = gamma[-1][:, None] * D1
    d1f = (d1 + beta[-1] @ D1).reshape(1, 32)

    full = lambda *shape: pl.BlockSpec(shape, lambda i: (0,) * len(shape))
    out = pl.pallas_call(
        _gnn_body,
        grid=(B // bc,),
        in_specs=[
            pl.BlockSpec((N, bc, 5), lambda i: (0, i, 0)),
            full(5, H), full(1, H),
            full(L, H, 2 * H), full(L, 4, H), full(L, H, H),
            full(L, 2, H),
            full(L, H, H), full(L, 1, H), full(L - 1, 1, H),
            full(H, 32), full(1, 32), full(32, 1), full(1, 1),
        ],
        out_specs=pl.BlockSpec((1, N, bc), lambda i: (i, 0, 0)),
        out_shape=jax.ShapeDtypeStruct((B // bc, N, bc), jnp.float32),
    )(xt, W_enc, b_enc.reshape(1, H),
      wfused, bdir, w2u, cu,
      U2, ccomb[:, None, :], gamma[:-1][:, None, :],
      d1g, d1f, D2, d2.reshape(1, 1))
    return out.transpose(0, 2, 1).reshape(B, N)
